# baseline (speedup 1.0000x reference)
import functools

import jax
import jax.numpy as jnp
from jax import lax
from jax.experimental import pallas as pl
from jax.experimental.pallas import tpu as pltpu
from jax.experimental.pallas import tpu_sc as plsc

NC = 2
NS = 16
NW = NC * NS

RS = 8
RD = 8
DMA_GROUPS = 66


def _make_gather(vocab, dim, n):
    assert n % NW == 0
    b_per_w = n // NW
    assert b_per_w % RS == 0 and b_per_w % RD == 0
    d_groups = min(DMA_GROUPS, b_per_w // RD)
    s_rows_total = b_per_w - d_groups * RD
    assert s_rows_total % RS == 0
    s_groups = s_rows_total // RS
    assert s_groups == 0 or s_groups >= 2
    assert d_groups == 0 or d_groups >= 2
    s_rows = s_groups * RS

    mesh = plsc.VectorSubcoreMesh(core_axis_name="c", subcore_axis_name="s")

    @functools.partial(
        pl.kernel,
        out_type=jax.ShapeDtypeStruct((n, dim), jnp.float32),
        mesh=mesh,
        scratch_types=[
            pltpu.VMEM((b_per_w + 16,), jnp.int32),
            [pltpu.VMEM((RS, dim), jnp.float32) for _ in range(2)],
            pltpu.VMEM_SHARED((NS, 2, RD, dim), jnp.float32),
            [pltpu.SemaphoreType.DMA for _ in range(2)],
            [pltpu.SemaphoreType.DMA for _ in range(2)],
            [pltpu.SemaphoreType.DMA for _ in range(2)],
            [pltpu.SemaphoreType.DMA for _ in range(2)],
        ],
    )
    def gather(table_hbm, idx_hbm, out_hbm, idx_v, sbufs, shared,
               sg_sems, ss_sems, dg_sems, ds_sems):
        cid = lax.axis_index("c")
        sid = lax.axis_index("s")
        wid = sid * NC + cid
        base = wid * b_per_w
        pltpu.sync_copy(idx_hbm.at[pl.ds(base, b_per_w)], idx_v.at[pl.ds(0, b_per_w)])

        def s_issue(g, slot):
            pltpu.async_copy(
                table_hbm.at[idx_v.at[pl.ds(g * RS, RS)]],
                sbufs[slot],
                sg_sems[slot],
            )

        def s_step(g):
            nxt = g + 1
            for slot in range(2):
                @pl.when(lax.rem(g, 2) == slot)
                def _():
                    other = 1 - slot
                    @pl.when(nxt < s_groups)
                    def _():
                        @pl.when(nxt >= 2)
                        def _():
                            pltpu.make_async_copy(
                                sbufs[other], out_hbm.at[pl.ds(base, RS)],
                                ss_sems[other],
                            ).wait()
                        s_issue(nxt, other)
                    pltpu.make_async_copy(
                        table_hbm.at[pl.ds(0, RS)], sbufs[slot], sg_sems[slot]
                    ).wait()
                    pltpu.async_copy(
                        sbufs[slot], out_hbm.at[pl.ds(base + g * RS, RS)],
                        ss_sems[slot],
                    )

        def d_issue(g, slot):
            vec = idx_v[pl.ds(s_rows + g * RD, 16)]
            for j in range(RD):
                row = vec[j]
                pltpu.async_copy(
                    table_hbm.at[pl.ds(row, 1)],
                    shared.at[sid, slot, pl.ds(j, 1)],
                    dg_sems[slot],
                )

        def d_step(g):
            nxt = g + 1
            for slot in range(2):
                @pl.when(lax.rem(g, 2) == slot)
                def _():
                    other = 1 - slot
                    @pl.when(nxt < d_groups)
                    def _():
                        @pl.when(nxt >= 2)
                        def _():
                            pltpu.make_async_copy(
                                shared.at[sid, other],
                                out_hbm.at[pl.ds(base, RD)],
                                ds_sems[other],
                            ).wait()
                        d_issue(nxt, other)
                    pltpu.make_async_copy(
                        table_hbm.at[pl.ds(0, RD)], shared.at[sid, slot],
                        dg_sems[slot],
                    ).wait()
                    pltpu.async_copy(
                        shared.at[sid, slot],
                        out_hbm.at[pl.ds(base + s_rows + g * RD, RD)],
                        ds_sems[slot],
                    )

        if s_groups > 0:
            s_issue(0, 0)
        if d_groups > 0:
            d_issue(0, 0)

        def body(t, _):
            if s_groups > 0:
                @pl.when(t < s_groups)
                def _():
                    s_step(t)
            if d_groups > 0:
                @pl.when(t < d_groups)
                def _():
                    d_step(t)
            return 0

        lax.fori_loop(0, max(s_groups, d_groups), body, 0)

        for slot in range(2):
            if s_groups > 0:
                pltpu.make_async_copy(
                    sbufs[slot], out_hbm.at[pl.ds(base, RS)], ss_sems[slot]
                ).wait()
            if d_groups > 0:
                pltpu.make_async_copy(
                    shared.at[sid, slot], out_hbm.at[pl.ds(base, RD)],
                    ds_sems[slot],
                ).wait()

    return gather


def kernel(input_ids, weight):
    b, s = input_ids.shape
    vocab, dim = weight.shape
    idx = input_ids.reshape(-1).astype(jnp.int32)
    out = _make_gather(vocab, dim, idx.shape[0])(weight, idx)
    return out.reshape(b, s, dim)
